# baseline (speedup 1.0000x reference)
import functools

import jax
import jax.numpy as jnp
from jax.experimental import pallas as pl


def _cch_kernel(q_ref, k_ref, m_ref, pdw_ref, out_ref, *, inv_bp, inv_bpd):
    b = pl.program_id(0)
    q = q_ref[0]
    k = k_ref[0]
    qq = jnp.sum(q * q, axis=1, keepdims=True)
    kk = jnp.sum(k * k, axis=1, keepdims=True)
    qk = jnp.dot(q, k.T, preferred_element_type=jnp.float32)
    d = qq + kk.T - 2.0 * qk
    cham_x = jnp.min(d, axis=1)
    cham_y = jnp.min(d, axis=0)
    m = m_ref[0, 0]
    pdw = pdw_ref[0]
    part = (jnp.sum(cham_x) + jnp.sum(cham_y * m)) * inv_bp
    part = part + jnp.sum(pdw * pdw) * inv_bpd

    @pl.when(b == 0)
    def _():
        out_ref[...] = jnp.zeros_like(out_ref)

    out_ref[...] += part[None, None]


def kernel(v, v_pred, mask, pred_dw):
    B, P, D = v.shape
    mask_flat = mask.reshape(B, 1, P)
    kern = functools.partial(
        _cch_kernel, inv_bp=1.0 / (B * P), inv_bpd=1.0 / (B * P * D)
    )
    out = pl.pallas_call(
        kern,
        grid=(B,),
        in_specs=[
            pl.BlockSpec((1, P, D), lambda b: (b, 0, 0)),
            pl.BlockSpec((1, P, D), lambda b: (b, 0, 0)),
            pl.BlockSpec((1, 1, P), lambda b: (b, 0, 0)),
            pl.BlockSpec((1, P, D), lambda b: (b, 0, 0)),
        ],
        out_specs=pl.BlockSpec((1, 1), lambda b: (0, 0)),
        out_shape=jax.ShapeDtypeStruct((1, 1), jnp.float32),
    )(v_pred, v, mask_flat, pred_dw)
    return out[0, 0]
